# baseline (speedup 1.0000x reference)
import functools

import jax
import jax.numpy as jnp
import numpy as np
from jax import lax
from jax.experimental import pallas as pl
from jax.experimental.pallas import tpu as pltpu
from jax.experimental.pallas import tpu_sc as plsc

B, S, V, D = 4096, 200, 1000000, 64
NC, NS = 2, 16
NW = NC * NS
ROWS_PER_W = B // NW
LANES = 16
SPLIT = 128


def _positional_encoding() -> np.ndarray:
    pos = np.arange(S, dtype=np.float32)[:, None]
    i = np.arange(D, dtype=np.float32)[None, :]
    angle_rates = 1.0 / np.power(10000.0, (2.0 * np.floor(i / 2.0)) / np.float32(D))
    angle_rads = pos * angle_rates
    pe = np.zeros((S, D), dtype=np.float32)
    pe[:, 0::2] = np.sin(angle_rads[:, 0::2])
    pe[:, 1::2] = np.cos(angle_rads[:, 1::2])
    return pe


_PE = _positional_encoding()


def kernel(inputs, table):
    idx_flat = inputs.reshape(B * S)
    pe = jnp.asarray(_PE)

    mesh = plsc.VectorSubcoreMesh(core_axis_name="c", subcore_axis_name="s")

    @functools.partial(
        pl.kernel,
        out_type=jax.ShapeDtypeStruct((B * S, D), jnp.float32),
        mesh=mesh,
        compiler_params=pltpu.CompilerParams(use_tc_tiling_on_sc=False),
        scratch_types=[
            pltpu.VMEM((SPLIT,), jnp.int32),
            pltpu.VMEM((S - SPLIT,), jnp.int32),
            pltpu.VMEM((S, D), jnp.float32),
            pltpu.VMEM((S, D), jnp.float32),
            pltpu.SemaphoreType.DMA,
        ],
    )
    def run(idx_hbm, table_hbm, pe_hbm, out_hbm, idx_a, idx_b, rows_v, pe_v, sem):
        wid = lax.axis_index("s") * NC + lax.axis_index("c")
        pltpu.sync_copy(pe_hbm, pe_v)

        @pl.loop(0, ROWS_PER_W)
        def _row(r):
            base = (wid * ROWS_PER_W + r) * S
            pltpu.sync_copy(idx_hbm.at[pl.ds(base, SPLIT)], idx_a)
            pltpu.sync_copy(idx_hbm.at[pl.ds(base + SPLIT, S - SPLIT)], idx_b)
            ga = pltpu.async_copy(table_hbm.at[idx_a], rows_v.at[pl.ds(0, SPLIT)], sem)
            gb = pltpu.async_copy(
                table_hbm.at[idx_b], rows_v.at[pl.ds(SPLIT, S - SPLIT)], sem)
            ga.wait()
            gb.wait()

            @pl.loop(0, S)
            def _add(i):
                for j in range(D // LANES):
                    sl = (i, pl.ds(j * LANES, LANES))
                    rows_v[sl] = rows_v[sl] + pe_v[sl]

            pltpu.sync_copy(rows_v, out_hbm.at[pl.ds(base, S)])

    out = run(idx_flat, table, pe)
    return out.reshape(B, S, D)


# speedup vs baseline: 1.2482x; 1.2482x over previous
import functools

import jax
import jax.numpy as jnp
import numpy as np
from jax import lax
from jax.experimental import pallas as pl
from jax.experimental.pallas import tpu as pltpu
from jax.experimental.pallas import tpu_sc as plsc

B, S, V, D = 4096, 200, 1000000, 64
NC, NS = 2, 16
NW = NC * NS
ROWS_PER_W = B // NW
LANES = 16
SPLIT = 128
NSLOT = 4
ADD_STEP = 8


def _positional_encoding() -> np.ndarray:
    pos = np.arange(S, dtype=np.float32)[:, None]
    i = np.arange(D, dtype=np.float32)[None, :]
    angle_rates = 1.0 / np.power(10000.0, (2.0 * np.floor(i / 2.0)) / np.float32(D))
    angle_rads = pos * angle_rates
    pe = np.zeros((S, D), dtype=np.float32)
    pe[:, 0::2] = np.sin(angle_rads[:, 0::2])
    pe[:, 1::2] = np.cos(angle_rads[:, 1::2])
    return pe


_PE = _positional_encoding()


def kernel(inputs, table):
    idx_flat = inputs.reshape(B * S)
    pe = jnp.asarray(_PE)

    mesh = plsc.VectorSubcoreMesh(core_axis_name="c", subcore_axis_name="s")

    @functools.partial(
        pl.kernel,
        out_type=jax.ShapeDtypeStruct((B * S, D), jnp.float32),
        mesh=mesh,
        compiler_params=pltpu.CompilerParams(use_tc_tiling_on_sc=False),
        scratch_types=[
            pltpu.VMEM((ROWS_PER_W * S,), jnp.int32),
            pltpu.VMEM((S, D), jnp.float32),
            pltpu.VMEM((S, D), jnp.float32),
            pltpu.VMEM((S, D), jnp.float32),
            pltpu.VMEM((S, D), jnp.float32),
            pltpu.VMEM((S, D), jnp.float32),
            pltpu.SemaphoreType.DMA,
            pltpu.SemaphoreType.DMA,
            pltpu.SemaphoreType.DMA,
            pltpu.SemaphoreType.DMA,
            pltpu.SemaphoreType.DMA,
            pltpu.SemaphoreType.DMA,
            pltpu.SemaphoreType.DMA,
            pltpu.SemaphoreType.DMA,
        ],
    )
    def run(idx_hbm, table_hbm, pe_hbm, out_hbm,
            idx_v, pe_v, rows0, rows1, rows2, rows3,
            g0, g1, g2, g3, w0, w1, w2, w3):
        wid = lax.axis_index("s") * NC + lax.axis_index("c")
        wbase = wid * ROWS_PER_W * S
        pltpu.sync_copy(idx_hbm.at[pl.ds(wbase, ROWS_PER_W * S)], idx_v)
        pltpu.sync_copy(pe_hbm, pe_v)

        rows = (rows0, rows1, rows2, rows3)
        gsem = (g0, g1, g2, g3)
        wsem = (w0, w1, w2, w3)

        def gather(m, k):
            off = m * S
            a = pltpu.make_async_copy(
                table_hbm.at[idx_v.at[pl.ds(off, SPLIT)]],
                rows[k].at[pl.ds(0, SPLIT)], gsem[k])
            b = pltpu.make_async_copy(
                table_hbm.at[idx_v.at[pl.ds(off + SPLIT, S - SPLIT)]],
                rows[k].at[pl.ds(SPLIT, S - SPLIT)], gsem[k])
            return a, b

        def gather_start(m, k):
            a, b = gather(m, k)
            a.start()
            b.start()

        def gather_wait(m, k):
            a, b = gather(m, k)
            a.wait()
            b.wait()

        def wb(m, k):
            return pltpu.make_async_copy(
                rows[k], out_hbm.at[pl.ds(wbase + m * S, S)], wsem[k])

        def add_pe(k):
            @pl.loop(0, S, step=ADD_STEP)
            def _add(i):
                for di in range(ADD_STEP):
                    for j in range(D // LANES):
                        sl = (i + di, pl.ds(j * LANES, LANES))
                        rows[k][sl] = rows[k][sl] + pe_v[sl]

        def process(m, k):
            gather_wait(m, k)
            add_pe(k)
            wb(m, k).start()

        gather_start(0, 0)
        gather_start(1, 1)
        gather_start(2, 2)
        process(0, 0)
        gather_start(3, 3)
        process(1, 1)
        wb(0, 0).wait()
        gather_start(4, 0)
        process(2, 2)
        wb(1, 1).wait()
        gather_start(5, 1)
        process(3, 3)

        @pl.loop(NSLOT, ROWS_PER_W - NSLOT, step=NSLOT)
        def _body(r):
            wb(r - 2, 2).wait()
            gather_start(r + 2, 2)
            process(r, 0)
            wb(r - 1, 3).wait()
            gather_start(r + 3, 3)
            process(r + 1, 1)
            wb(r, 0).wait()
            gather_start(r + 4, 0)
            process(r + 2, 2)
            wb(r + 1, 1).wait()
            gather_start(r + 5, 1)
            process(r + 3, 3)

        E = ROWS_PER_W - NSLOT
        wb(E - 2, 2).wait()
        gather_start(E + 2, 2)
        process(E, 0)
        wb(E - 1, 3).wait()
        gather_start(E + 3, 3)
        process(E + 1, 1)
        process(E + 2, 2)
        process(E + 3, 3)
        wb(E, 0).wait()
        wb(E + 1, 1).wait()
        wb(E + 2, 2).wait()
        wb(E + 3, 3).wait()

    out = run(idx_flat, table, pe)
    return out.reshape(B, S, D)
